# Initial kernel scaffold; baseline (speedup 1.0000x reference)
import jax
import jax.numpy as jnp
from jax.experimental import pallas as pl


def kernel(x, token_table, pos_table, ln_w, ln_b):
    raise NotImplementedError("write your pallas kernel here")


# speedup vs baseline: 1.5647x; 1.5647x over previous
import functools

import jax
import jax.numpy as jnp
from jax import lax
from jax.experimental import pallas as pl
from jax.experimental.pallas import tpu as pltpu
import jax.experimental.pallas.tpu_sc as plsc

EPS = 1e-5


def _sc_gather(table, flat_idx):
    n, = flat_idx.shape
    _, d = table.shape
    info = plsc.get_sparse_core_info()
    nw = info.num_cores * info.num_subcores
    per_w = n // nw
    chunk = 128
    n_chunks = per_w // chunk
    mesh = plsc.VectorSubcoreMesh(core_axis_name="c", subcore_axis_name="s")

    @functools.partial(
        pl.kernel,
        mesh=mesh,
        out_type=jax.ShapeDtypeStruct((n, d), jnp.float32),
        scratch_types=[
            pltpu.VMEM((chunk,), jnp.int32),
            pltpu.VMEM((chunk, d), jnp.float32),
            pltpu.SemaphoreType.DMA,
        ],
    )
    def gather_kernel(table_hbm, idx_hbm, out_hbm, idx_v, rows_v, sem):
        wid = lax.axis_index("s") * info.num_cores + lax.axis_index("c")

        def body(i, carry):
            base = wid * per_w + i * chunk
            pltpu.sync_copy(idx_hbm.at[pl.ds(base, chunk)], idx_v)
            pltpu.async_copy(table_hbm.at[idx_v], rows_v, sem).wait()
            pltpu.sync_copy(rows_v, out_hbm.at[pl.ds(base, chunk)])
            return carry

        lax.fori_loop(0, n_chunks, body, 0)

    return gather_kernel(table, flat_idx)


def _tc_pos_ln(tok, pos_table, ln_w, ln_b):
    b, l, d = tok.shape
    bb = 64
    grid = (b // bb,)

    def body(tok_ref, pos_ref, w_ref, b_ref, out_ref):
        e = tok_ref[...] + pos_ref[...][None]
        m = jnp.mean(e, axis=-1, keepdims=True)
        c = e - m
        v = jnp.mean(c * c, axis=-1, keepdims=True)
        out_ref[...] = (c * lax.rsqrt(v + EPS)) * w_ref[...] + b_ref[...]

    return pl.pallas_call(
        body,
        grid=grid,
        in_specs=[
            pl.BlockSpec((bb, l, d), lambda i: (i, 0, 0)),
            pl.BlockSpec((l, d), lambda i: (0, 0)),
            pl.BlockSpec((d,), lambda i: (0,)),
            pl.BlockSpec((d,), lambda i: (0,)),
        ],
        out_specs=pl.BlockSpec((bb, l, d), lambda i: (i, 0, 0)),
        out_shape=jax.ShapeDtypeStruct((b, l, d), jnp.float32),
    )(tok, pos_table, ln_w, ln_b)


@jax.jit
def kernel(x, token_table, pos_table, ln_w, ln_b):
    b, l = x.shape
    flat = x.reshape(-1).astype(jnp.int32)
    tok = _sc_gather(token_table, flat)
    return _tc_pos_ln(tok.reshape(b, l, -1), pos_table, ln_w, ln_b)


# speedup vs baseline: 2.2162x; 1.4164x over previous
import functools

import jax
import jax.numpy as jnp
from jax import lax
from jax.experimental import pallas as pl
from jax.experimental.pallas import tpu as pltpu
import jax.experimental.pallas.tpu_sc as plsc

EPS = 1e-5


def _sc_gather(table, flat_idx):
    n, = flat_idx.shape
    _, d = table.shape
    info = plsc.get_sparse_core_info()
    nw = info.num_cores * info.num_subcores
    per_w = n // nw
    chunk = 128
    nbuf = 4
    n_iters = per_w // (chunk * nbuf)
    mesh = plsc.VectorSubcoreMesh(core_axis_name="c", subcore_axis_name="s")

    @functools.partial(
        pl.kernel,
        mesh=mesh,
        out_type=jax.ShapeDtypeStruct((n, d), jnp.float32),
        scratch_types=[
            pltpu.VMEM((nbuf, chunk), jnp.int32),
            pltpu.VMEM((nbuf, chunk, d), jnp.float32),
            pltpu.SemaphoreType.DMA((nbuf,)),
            pltpu.SemaphoreType.DMA((nbuf,)),
        ],
    )
    def gather_kernel(table_hbm, idx_hbm, out_hbm, idx_v, rows_v, gsem, ssem):
        wid = lax.axis_index("s") * info.num_cores + lax.axis_index("c")
        w_base = wid * per_w

        def body(i, carry):
            for b in range(nbuf):
                base = w_base + (i * nbuf + b) * chunk

                @pl.when(i > 0)
                def _():
                    prev = w_base + ((i - 1) * nbuf + b) * chunk
                    pltpu.make_async_copy(
                        rows_v.at[b], out_hbm.at[pl.ds(prev, chunk)],
                        ssem.at[b]).wait()

                pltpu.sync_copy(idx_hbm.at[pl.ds(base, chunk)], idx_v.at[b])
                pltpu.async_copy(
                    table_hbm.at[idx_v.at[b]], rows_v.at[b], gsem.at[b])
            for b in range(nbuf):
                base = w_base + (i * nbuf + b) * chunk
                pltpu.make_async_copy(
                    table_hbm.at[idx_v.at[b]], rows_v.at[b], gsem.at[b]).wait()
                pltpu.async_copy(
                    rows_v.at[b], out_hbm.at[pl.ds(base, chunk)], ssem.at[b])
            return carry

        lax.fori_loop(0, n_iters, body, 0)
        for b in range(nbuf):
            last = w_base + ((n_iters - 1) * nbuf + b) * chunk
            pltpu.make_async_copy(
                rows_v.at[b], out_hbm.at[pl.ds(last, chunk)], ssem.at[b]).wait()

    return gather_kernel(table, flat_idx)


def _tc_pos_ln(tok, pos_table, ln_w, ln_b):
    b, l, d = tok.shape
    bb = 64
    grid = (b // bb,)

    def body(tok_ref, pos_ref, w_ref, b_ref, out_ref):
        e = tok_ref[...] + pos_ref[...][None]
        m = jnp.mean(e, axis=-1, keepdims=True)
        c = e - m
        v = jnp.mean(c * c, axis=-1, keepdims=True)
        out_ref[...] = (c * lax.rsqrt(v + EPS)) * w_ref[...] + b_ref[...]

    return pl.pallas_call(
        body,
        grid=grid,
        in_specs=[
            pl.BlockSpec((bb, l, d), lambda i: (i, 0, 0)),
            pl.BlockSpec((l, d), lambda i: (0, 0)),
            pl.BlockSpec((d,), lambda i: (0,)),
            pl.BlockSpec((d,), lambda i: (0,)),
        ],
        out_specs=pl.BlockSpec((bb, l, d), lambda i: (i, 0, 0)),
        out_shape=jax.ShapeDtypeStruct((b, l, d), jnp.float32),
    )(tok, pos_table, ln_w, ln_b)


@jax.jit
def kernel(x, token_table, pos_table, ln_w, ln_b):
    b, l = x.shape
    flat = x.reshape(-1).astype(jnp.int32)
    tok = _sc_gather(token_table, flat)
    return _tc_pos_ln(tok.reshape(b, l, -1), pos_table, ln_w, ln_b)
